# baseline (speedup 1.0000x reference)
import functools

import jax
import jax.numpy as jnp
from jax import lax
from jax.experimental import pallas as pl
from jax.experimental.pallas import tpu as pltpu
from jax.experimental.pallas import tpu_sc as plsc

_LANES = 16


def _make_sc_kernel(B, T, D, NW, NC):
    bw = B // NW
    groups = D // _LANES
    mesh = plsc.VectorSubcoreMesh(core_axis_name="c", subcore_axis_name="s")

    @functools.partial(
        pl.kernel,
        mesh=mesh,
        out_type=jax.ShapeDtypeStruct((B, T * D), jnp.float32),
        scratch_types=[
            pltpu.VMEM((T, bw), jnp.int32),
            pltpu.VMEM((T, D), jnp.float32),
            pltpu.VMEM((bw, D), jnp.float32),
            pltpu.SemaphoreType.DMA,
        ],
    )
    def sc_kernel(tok_hbm, table_hbm, pos_hbm, out_hbm, idx_v, pos_v, rows_v, sem):
        c = lax.axis_index("c")
        s = lax.axis_index("s")
        w = s * NC + c
        base = w * bw
        pltpu.sync_copy(tok_hbm.at[w], idx_v)
        pltpu.sync_copy(pos_hbm, pos_v)

        def step(t, _):
            pltpu.async_copy(table_hbm.at[idx_v.at[t]], rows_v, sem).wait()
            for g in range(groups):
                pv = pos_v[t, pl.ds(g * _LANES, _LANES)]
                for r in range(bw):
                    sl = (r, pl.ds(g * _LANES, _LANES))
                    rows_v[sl] = rows_v[sl] + pv
            pltpu.sync_copy(rows_v, out_hbm.at[pl.ds(base, bw), pl.ds(t * D, D)])
            return ()

        lax.fori_loop(0, T, step, (), unroll=False)

    return sc_kernel


def kernel(tokens, token_table, position_embedding):
    B, T = tokens.shape
    V, D = token_table.shape
    NW = 32
    NC = 2
    assert B % NW == 0 and D % _LANES == 0
    tok = tokens.astype(jnp.int32).reshape(NW, B // NW, T).transpose(0, 2, 1)
    sc = _make_sc_kernel(B, T, D, NW, NC)
    out2d = sc(tok, token_table, position_embedding)
    return out2d.reshape(B, T, D)


# speedup vs baseline: 1.1682x; 1.1682x over previous
import functools

import jax
import jax.numpy as jnp
from jax import lax
from jax.experimental import pallas as pl
from jax.experimental.pallas import tpu as pltpu
from jax.experimental.pallas import tpu_sc as plsc

_LANES = 16


def _make_sc_kernel(B, T, D, NW, NC):
    bw = B // NW
    mesh = plsc.VectorSubcoreMesh(core_axis_name="c", subcore_axis_name="s")

    @functools.partial(
        pl.kernel,
        mesh=mesh,
        compiler_params=pltpu.CompilerParams(needs_layout_passes=False),
        out_type=jax.ShapeDtypeStruct((B, T * D), jnp.float32),
        scratch_types=[
            pltpu.VMEM((bw * T,), jnp.int32),
            pltpu.VMEM((T, bw), jnp.int32),
            pltpu.VMEM((T, D), jnp.float32),
            pltpu.VMEM((bw, D), jnp.float32),
            pltpu.VMEM((bw, D), jnp.float32),
            pltpu.SemaphoreType.DMA,
            pltpu.SemaphoreType.DMA,
            pltpu.SemaphoreType.DMA,
            pltpu.SemaphoreType.DMA,
        ],
    )
    def sc_kernel(tok_hbm, table_hbm, pos_hbm, out_hbm,
                  tokw_v, idx_v, pos_v, buf0, buf1, gsem0, gsem1, ssem0, ssem1):
        c = lax.axis_index("c")
        s = lax.axis_index("s")
        w = s * NC + c
        base = w * bw
        pltpu.sync_copy(tok_hbm.at[pl.ds(base * T, bw * T)], tokw_v)
        pltpu.sync_copy(pos_hbm, pos_v)

        lane = lax.iota(jnp.int32, _LANES)

        def transpose_step(t, _):
            flat = lane * T + t
            for j0 in range(0, bw, _LANES):
                col = plsc.load_gather(tokw_v, [j0 * T + flat])
                idx_v[t, pl.ds(j0, _LANES)] = col
            return ()

        lax.fori_loop(0, T, transpose_step, (), unroll=False)

        def start_gather(t, buf, sem):
            pltpu.async_copy(table_hbm.at[idx_v.at[t]], buf, sem)

        def wait_gather(buf, sem):
            pltpu.make_async_copy(table_hbm.at[pl.ds(0, bw), :], buf, sem).wait()

        def out_slice(t):
            off = pl.multiple_of(t * D, D)
            return out_hbm.at[pl.ds(base, bw), pl.ds(off, D)]

        def start_scatter(t, buf, sem):
            pltpu.async_copy(buf, out_slice(t), sem)

        def wait_scatter(buf, sem):
            pltpu.make_async_copy(buf, out_slice(0), sem).wait()

        def add_pos(t, buf):
            for g in range(D // _LANES):
                sl = pl.ds(g * _LANES, _LANES)
                pv = pos_v[t, sl]
                for r in range(bw):
                    plsc.addupdate(buf.at[r, sl], pv)

        start_gather(0, buf0, gsem0)

        @pl.loop(0, T - 1, step=2)
        def pair(t0):
            @pl.when(t0 > 0)
            def _():
                wait_scatter(buf1, ssem1)
            start_gather(t0 + 1, buf1, gsem1)
            wait_gather(buf0, gsem0)
            add_pos(t0, buf0)
            start_scatter(t0, buf0, ssem0)
            wait_scatter(buf0, ssem0)
            start_gather(t0 + 2, buf0, gsem0)
            wait_gather(buf1, gsem1)
            add_pos(t0 + 1, buf1)
            start_scatter(t0 + 1, buf1, ssem1)

        wait_scatter(buf1, ssem1)
        wait_gather(buf0, gsem0)
        add_pos(T - 1, buf0)
        start_scatter(T - 1, buf0, ssem0)
        wait_scatter(buf0, ssem0)

    return sc_kernel


def kernel(tokens, token_table, position_embedding):
    B, T = tokens.shape
    V, D = token_table.shape
    NW = 32
    NC = 2
    assert B % NW == 0 and D % _LANES == 0 and T % 2 == 1
    sc = _make_sc_kernel(B, T, D, NW, NC)
    out2d = sc(tokens.astype(jnp.int32).reshape(B * T), token_table,
               position_embedding)
    return out2d.reshape(B, T, D)


# speedup vs baseline: 1.3337x; 1.1417x over previous
import functools

import jax
import jax.numpy as jnp
from jax import lax
from jax.experimental import pallas as pl
from jax.experimental.pallas import tpu as pltpu
from jax.experimental.pallas import tpu_sc as plsc

_LANES = 16


def _make_sc_kernel(B, T, D, NW, NC):
    bw = B // NW
    Tp = (T + 7) // 8 * 8
    Th = Tp // 2
    Tr = T - Th
    nchunks = 2 * bw
    mesh = plsc.VectorSubcoreMesh(core_axis_name="c", subcore_axis_name="s")

    @functools.partial(
        pl.kernel,
        mesh=mesh,
        out_type=jax.ShapeDtypeStruct((B * Tp, D), jnp.float32),
        scratch_types=[
            pltpu.VMEM((bw, Tp), jnp.int32),
            pltpu.VMEM((T, D), jnp.float32),
            pltpu.VMEM((Th, D), jnp.float32),
            pltpu.VMEM((Th, D), jnp.float32),
            pltpu.SemaphoreType.DMA,
            pltpu.SemaphoreType.DMA,
            pltpu.SemaphoreType.DMA,
            pltpu.SemaphoreType.DMA,
        ],
    )
    def sc_kernel(tok_hbm, table_hbm, pos_hbm, out_hbm,
                  tokw_v, pos_v, buf0, buf1, gsem0, gsem1, ssem0, ssem1):
        c = lax.axis_index("c")
        s = lax.axis_index("s")
        w = s * NC + c
        base = w * bw
        pltpu.sync_copy(tok_hbm.at[pl.ds(base, bw), :], tokw_v)
        pltpu.sync_copy(pos_hbm, pos_v)

        def start_gather(j, h, buf, sem):
            idx = tokw_v.at[j, pl.ds(h * Th, Th)]
            pltpu.async_copy(table_hbm.at[idx], buf, sem)

        def wait_gather(buf, sem):
            pltpu.make_async_copy(table_hbm.at[pl.ds(0, Th), :], buf,
                                  sem).wait()

        def start_scatter(j, h, buf, sem):
            row0 = (base + j) * Tp + h * Th
            pltpu.async_copy(buf, out_hbm.at[pl.ds(row0, Th), :], sem)

        def wait_scatter(buf, sem):
            pltpu.make_async_copy(buf, out_hbm.at[pl.ds(0, Th), :],
                                  sem).wait()

        def add_pos(h, buf):
            nrows = Th if h == 0 else Tr

            def row(r, _):
                for g in range(D // _LANES):
                    sl = pl.ds(g * _LANES, _LANES)
                    plsc.addupdate(buf.at[r, sl], pos_v[h * Th + r, sl])
                return ()

            lax.fori_loop(0, nrows, row, (), unroll=False)

        start_gather(0, 0, buf0, gsem0)

        @pl.loop(0, nchunks - 2, step=2)
        def pair(k0):
            j = k0 // 2
            @pl.when(k0 > 0)
            def _():
                wait_scatter(buf1, ssem1)
            start_gather(j, 1, buf1, gsem1)
            wait_gather(buf0, gsem0)
            add_pos(0, buf0)
            start_scatter(j, 0, buf0, ssem0)
            wait_scatter(buf0, ssem0)
            start_gather(j + 1, 0, buf0, gsem0)
            wait_gather(buf1, gsem1)
            add_pos(1, buf1)
            start_scatter(j, 1, buf1, ssem1)

        j = bw - 1
        wait_scatter(buf1, ssem1)
        start_gather(j, 1, buf1, gsem1)
        wait_gather(buf0, gsem0)
        add_pos(0, buf0)
        start_scatter(j, 0, buf0, ssem0)
        wait_gather(buf1, gsem1)
        add_pos(1, buf1)
        start_scatter(j, 1, buf1, ssem1)
        wait_scatter(buf0, ssem0)
        wait_scatter(buf1, ssem1)

    return sc_kernel


def kernel(tokens, token_table, position_embedding):
    B, T = tokens.shape
    V, D = token_table.shape
    NW = 32
    NC = 2
    assert B % NW == 0 and D % _LANES == 0
    Tp = (T + 7) // 8 * 8
    tok = jnp.pad(tokens.astype(jnp.int32), ((0, 0), (0, Tp - T)))
    sc = _make_sc_kernel(B, T, D, NW, NC)
    out2d = sc(tok, token_table, position_embedding)
    return out2d.reshape(B, Tp, D)[:, :T, :]


# speedup vs baseline: 1.3813x; 1.0356x over previous
import functools

import jax
import jax.numpy as jnp
from jax import lax
from jax.experimental import pallas as pl
from jax.experimental.pallas import tpu as pltpu
from jax.experimental.pallas import tpu_sc as plsc

_LANES = 16


def _make_sc_kernel(B, T, D, NW, NC):
    bw = B // NW
    Tp = (T + 7) // 8 * 8
    Th = Tp // 2
    Tr = T - Th
    nchunks = 2 * bw
    mesh = plsc.VectorSubcoreMesh(core_axis_name="c", subcore_axis_name="s")

    @functools.partial(
        pl.kernel,
        mesh=mesh,
        compiler_params=pltpu.CompilerParams(disable_bounds_checks=True),
        out_type=jax.ShapeDtypeStruct((B, T, D), jnp.float32),
        scratch_types=[
            pltpu.VMEM((bw, Tp), jnp.int32),
            pltpu.VMEM((T, D), jnp.float32),
            pltpu.VMEM((Th, D), jnp.float32),
            pltpu.VMEM((Th, D), jnp.float32),
            pltpu.SemaphoreType.DMA,
            pltpu.SemaphoreType.DMA,
            pltpu.SemaphoreType.DMA,
            pltpu.SemaphoreType.DMA,
        ],
    )
    def sc_kernel(tok_hbm, table_hbm, pos_hbm, out_hbm,
                  tokw_v, pos_v, buf0, buf1, gsem0, gsem1, ssem0, ssem1):
        c = lax.axis_index("c")
        s = lax.axis_index("s")
        w = s * NC + c
        base = w * bw
        pltpu.sync_copy(tok_hbm.at[pl.ds(base, bw), :], tokw_v)
        pltpu.sync_copy(pos_hbm, pos_v)

        def start_gather(j, h, buf, sem):
            idx = tokw_v.at[j, pl.ds(h * Th, Th)]
            pltpu.async_copy(table_hbm.at[idx], buf, sem)

        def wait_gather(buf, sem):
            pltpu.make_async_copy(table_hbm.at[pl.ds(0, Th), :], buf,
                                  sem).wait()

        def start_scatter(j, h, buf, sem):
            off = pl.multiple_of(h * Th + w * 0, Th)
            pltpu.async_copy(buf, out_hbm.at[base + j, pl.ds(off, Th), :],
                             sem)

        def wait_scatter(buf, sem):
            pltpu.make_async_copy(buf, out_hbm.at[base, pl.ds(0, Th), :],
                                  sem).wait()

        def add_pos(h, buf):
            nrows = Th if h == 0 else Tr

            def row(r, _):
                for g in range(D // _LANES):
                    sl = pl.ds(g * _LANES, _LANES)
                    plsc.addupdate(buf.at[r, sl], pos_v[h * Th + r, sl])
                return ()

            lax.fori_loop(0, nrows, row, (), unroll=False)

        start_gather(0, 0, buf0, gsem0)

        @pl.loop(0, nchunks - 2, step=2)
        def pair(k0):
            j = k0 // 2
            @pl.when(k0 > 0)
            def _():
                wait_scatter(buf1, ssem1)
            start_gather(j, 1, buf1, gsem1)
            wait_gather(buf0, gsem0)
            add_pos(0, buf0)
            start_scatter(j, 0, buf0, ssem0)
            wait_scatter(buf0, ssem0)
            start_gather(j + 1, 0, buf0, gsem0)
            wait_gather(buf1, gsem1)
            add_pos(1, buf1)
            start_scatter(j, 1, buf1, ssem1)

        j = bw - 1
        wait_scatter(buf1, ssem1)
        start_gather(j, 1, buf1, gsem1)
        wait_gather(buf0, gsem0)
        add_pos(0, buf0)
        start_scatter(j, 0, buf0, ssem0)
        wait_gather(buf1, gsem1)
        add_pos(1, buf1)
        start_scatter(j, 1, buf1, ssem1)
        wait_scatter(buf0, ssem0)
        wait_scatter(buf1, ssem1)

    return sc_kernel


def kernel(tokens, token_table, position_embedding):
    B, T = tokens.shape
    V, D = token_table.shape
    NW = 32
    NC = 2
    assert B % NW == 0 and D % _LANES == 0
    Tp = (T + 7) // 8 * 8
    tok = jnp.pad(tokens.astype(jnp.int32), ((0, 0), (0, Tp - T)))
    sc = _make_sc_kernel(B, T, D, NW, NC)
    return sc(tok, token_table, position_embedding)
